# baseline (speedup 1.0000x reference)
import functools

import jax
import jax.numpy as jnp
from jax import lax
from jax.experimental import pallas as pl
from jax.experimental.pallas import tpu as pltpu
from jax.experimental.pallas import tpu_sc as plsc

_NC = 2
_NS = 16
_LANES = 16


def _prep_body(h_ref, m_ref, hid_ref, mid_ref, hm_ref, cid_ref):
    hm_ref[...] = h_ref[...][:, None, :] + m_ref[...][None, :, :]
    nm = m_ref.shape[0]
    cid_ref[...] = hid_ref[...] * nm + mid_ref[...]


def _prep(hour_table, minute_table, hour_ids, minute_ids):
    nh, d = hour_table.shape
    nm = minute_table.shape[0]
    hm, cid = pl.pallas_call(
        _prep_body,
        out_shape=(
            jax.ShapeDtypeStruct((nh, nm, d), jnp.float32),
            jax.ShapeDtypeStruct(hour_ids.shape, jnp.int32),
        ),
    )(hour_table, minute_table, hour_ids, minute_ids)
    return hm.reshape(nh * nm, d), cid


def _make_sc_kernel(B, T, S, D, NH, NM):
    NW = _NC * _NS
    SG = 4
    PG = NW // SG
    SPW = S // SG
    BPW = B // PG
    NP = BPW * T
    DVEC = D // _LANES
    QUAD = 4
    NSLOT = 2
    NBUF = QUAD * NSLOT
    NQRT = 2
    HPW = SPW // NQRT
    HVEC = HPW // _LANES

    mesh = plsc.VectorSubcoreMesh(
        core_axis_name="c", subcore_axis_name="s",
        num_cores=_NC, num_subcores=_NS)

    @functools.partial(
        pl.kernel,
        out_type=jax.ShapeDtypeStruct((B, T, S, D), jnp.float32),
        mesh=mesh,
        compiler_params=pltpu.CompilerParams(needs_layout_passes=False),
        scratch_types=[
            pltpu.VMEM((SPW, D), jnp.float32),
            pltpu.VMEM((NH * NM, D), jnp.float32),
            [pltpu.VMEM((HPW, D), jnp.float32)] * NBUF,
            pltpu.VMEM((BPW, T, SPW), jnp.int32),
            [pltpu.SemaphoreType.DMA] * NBUF,
        ],
    )
    def sc_kernel(pos_hbm, cid_hbm, hm_hbm, out_hbm,
                  pos_v, hm_v, rows, c_v, osem):
        cax = lax.axis_index("c")
        sid = lax.axis_index("s")
        wid = sid * _NC + cax
        sg = wid % SG
        pg = wid // SG
        s0 = sg * SPW
        b0 = pg * BPW

        pltpu.sync_copy(hm_hbm, hm_v)
        pltpu.sync_copy(pos_hbm.at[pl.ds(s0, SPW), :], pos_v)
        for bl in range(BPW):
            pltpu.sync_copy(cid_hbm.at[b0 + bl, :, pl.ds(s0, SPW)], c_v.at[bl])

        def start_out(i, h, b):
            bl = i // T
            t = i % T
            pltpu.async_copy(
                rows[b],
                out_hbm.at[b0 + bl, t, pl.ds(s0 + h * HPW, HPW), :], osem[b])

        def wait_out(b):
            pltpu.make_async_copy(
                rows[b], out_hbm.at[0, 0, pl.ds(0, HPW), :], osem[b]).wait()

        def fill_pass(i0, h, slot, first):
            bufs = [slot * QUAD + q for q in range(QUAD)]

            for q in range(QUAD):
                @pl.when(jnp.logical_not(first))
                def _():
                    wait_out(bufs[q])

            @plsc.parallel_loop(0, HVEC, unroll=2)
            def fill_body(g):
                rv = []
                for q in range(QUAD):
                    i = i0 + q
                    rv.append(c_v[i // T, i % T,
                                  pl.ds(h * HPW + g * _LANES, _LANES)])
                pend = None
                for l in range(_LANES):
                    r = g * _LANES + l
                    rg = h * HPW + r
                    row = [rv[q][l] for q in range(QUAD)]
                    for j in range(DVEC):
                        sl = pl.ds(j * _LANES, _LANES)
                        p = pos_v[rg, sl]
                        vs = [hm_v[row[q], sl] for q in range(QUAD)]
                        if pend is not None:
                            pvs, pp, pr, psl = pend
                            for q in range(QUAD):
                                rows[bufs[q]][pr, psl] = pvs[q] + pp
                        pend = (vs, p, r, sl)
                pvs, pp, pr, psl = pend
                for q in range(QUAD):
                    rows[bufs[q]][pr, psl] = pvs[q] + pp

            for q in range(QUAD):
                start_out(i0 + q, h, bufs[q])

        def loop_body(it, carry):
            for k in range(NSLOT):
                p = it * NSLOT + k
                fill_pass((p // NQRT) * QUAD, p % NQRT, k, it == 0)
            return carry
        lax.fori_loop(0, NP // QUAD * NQRT // NSLOT, loop_body, 0)

        for b in range(NBUF):
            wait_out(b)

    return sc_kernel


def kernel(batch_size, total_length, position_ids, week_ids, day_ids,
           hour_ids, minute_ids, device, position_table, week_table,
           day_table, hour_table, minute_table):
    S, D = position_table.shape
    B = hour_ids.shape[0]
    T = hour_ids.shape[1] // S
    NH = hour_table.shape[0]
    NM = minute_table.shape[0]

    hm, cid = _prep(hour_table, minute_table, hour_ids, minute_ids)

    sc = _make_sc_kernel(B, T, S, D, NH, NM)
    return sc(position_table, cid.reshape(B, T, S), hm)


# speedup vs baseline: 2.1899x; 2.1899x over previous
import functools

import jax
import jax.numpy as jnp
from jax import lax
from jax.experimental import pallas as pl
from jax.experimental.pallas import tpu as pltpu
from jax.experimental.pallas import tpu_sc as plsc

_NC = 2
_NS = 16
_LANES = 16


def _prep_body(h_ref, m_ref, hid_ref, mid_ref, hm_ref, cid_ref):
    hm_ref[...] = h_ref[...][:, None, :] + m_ref[...][None, :, :]
    nm = m_ref.shape[0]
    cid_ref[...] = hid_ref[...] * nm + mid_ref[...]


def _prep(hour_table, minute_table, hour_ids, minute_ids):
    nh, d = hour_table.shape
    nm = minute_table.shape[0]
    hm, cid = pl.pallas_call(
        _prep_body,
        out_shape=(
            jax.ShapeDtypeStruct((nh, nm, d), jnp.float32),
            jax.ShapeDtypeStruct(hour_ids.shape, jnp.int32),
        ),
    )(hour_table, minute_table, hour_ids, minute_ids)
    return hm.reshape(nh * nm, d), cid


def _make_sc_kernel(B, T, S, D, NH, NM):
    NW = _NC * _NS
    SG = 4
    PG = NW // SG
    SPW = S // SG
    BPW = B // PG
    NP = BPW * T
    DVEC = D // _LANES
    QUAD = 4
    NSLOT = 2
    NBUF = QUAD * NSLOT
    NQRT = 2
    HPW = SPW // NQRT
    HVEC = HPW // _LANES

    mesh = plsc.VectorSubcoreMesh(
        core_axis_name="c", subcore_axis_name="s",
        num_cores=_NC, num_subcores=_NS)

    @functools.partial(
        pl.kernel,
        out_type=jax.ShapeDtypeStruct((B, T, S, D), jnp.float32),
        mesh=mesh,
        compiler_params=pltpu.CompilerParams(needs_layout_passes=False),
        scratch_types=[
            pltpu.VMEM((SPW, D), jnp.float32),
            pltpu.VMEM((NH * NM, D), jnp.float32),
            [pltpu.VMEM((HPW, D), jnp.float32)] * NBUF,
            pltpu.VMEM((BPW, T, SPW), jnp.int32),
            [pltpu.SemaphoreType.DMA] * NBUF,
        ],
    )
    def sc_kernel(pos_hbm, cid_hbm, hm_hbm, out_hbm,
                  pos_v, hm_v, rows, c_v, osem):
        cax = lax.axis_index("c")
        sid = lax.axis_index("s")
        wid = sid * _NC + cax
        sg = wid % SG
        pg = wid // SG
        s0 = sg * SPW
        b0 = pg * BPW

        pltpu.sync_copy(hm_hbm, hm_v)
        pltpu.sync_copy(pos_hbm.at[pl.ds(s0, SPW), :], pos_v)
        for bl in range(BPW):
            pltpu.sync_copy(cid_hbm.at[b0 + bl, :, pl.ds(s0, SPW)], c_v.at[bl])

        def start_out(i, h, b):
            bl = i // T
            t = i % T
            pltpu.async_copy(
                rows[b],
                out_hbm.at[b0 + bl, t, pl.ds(s0 + h * HPW, HPW), :], osem[b])

        def wait_out(b):
            pltpu.make_async_copy(
                rows[b], out_hbm.at[0, 0, pl.ds(0, HPW), :], osem[b]).wait()

        def fill_pass(i0, h, slot, first):
            bufs = [slot * QUAD + q for q in range(QUAD)]

            for q in range(QUAD):
                @pl.when(jnp.logical_not(first))
                def _():
                    wait_out(bufs[q])

            @plsc.parallel_loop(0, HVEC)
            def fill_body(g):
                rv = []
                for q in range(QUAD):
                    i = i0 + q
                    rv.append(c_v[i // T, i % T,
                                  pl.ds(h * HPW + g * _LANES, _LANES)])
                pend = None
                for l in range(_LANES):
                    r = g * _LANES + l
                    rg = h * HPW + r
                    row = [rv[q][l] for q in range(QUAD)]
                    for j in range(DVEC):
                        sl = pl.ds(j * _LANES, _LANES)
                        p = pos_v[rg, sl]
                        vs = []
                        for q in range(QUAD):
                            vs.append(hm_v[row[q], sl])
                            if pend is not None:
                                pvs, pp, pr, psl = pend
                                rows[bufs[q]][pr, psl] = pvs[q] + pp
                        pend = (vs, p, r, sl)
                pvs, pp, pr, psl = pend
                for q in range(QUAD):
                    rows[bufs[q]][pr, psl] = pvs[q] + pp

            for q in range(QUAD):
                start_out(i0 + q, h, bufs[q])

        def loop_body(it, carry):
            for k in range(NSLOT):
                p = it * NSLOT + k
                fill_pass((p // NQRT) * QUAD, p % NQRT, k, it == 0)
            return carry
        lax.fori_loop(0, NP // QUAD * NQRT // NSLOT, loop_body, 0)

        for b in range(NBUF):
            wait_out(b)

    return sc_kernel


def kernel(batch_size, total_length, position_ids, week_ids, day_ids,
           hour_ids, minute_ids, device, position_table, week_table,
           day_table, hour_table, minute_table):
    S, D = position_table.shape
    B = hour_ids.shape[0]
    T = hour_ids.shape[1] // S
    NH = hour_table.shape[0]
    NM = minute_table.shape[0]

    hm, cid = _prep(hour_table, minute_table, hour_ids, minute_ids)

    sc = _make_sc_kernel(B, T, S, D, NH, NM)
    return sc(position_table, cid.reshape(B, T, S), hm)
